# Initial kernel scaffold; baseline (speedup 1.0000x reference)
import jax
import jax.numpy as jnp
from jax.experimental import pallas as pl


def kernel(atomic_energy):
    raise NotImplementedError("write your pallas kernel here")


# speedup vs baseline: 101.5577x; 101.5577x over previous
import functools

import jax
import jax.numpy as jnp
from jax import lax
from jax.experimental import pallas as pl
from jax.experimental.pallas import tpu as pltpu
from jax.experimental.pallas import tpu_sc as plsc

N = 3200000
NC = 2
NS = 16
NW = NC * NS
CHUNK = N // NW
LANES = 16
VPW = CHUNK // LANES
UNROLL = 5
ITERS = VPW // UNROLL

_mesh = plsc.VectorSubcoreMesh(core_axis_name="c", subcore_axis_name="s")


@functools.partial(
    pl.kernel,
    out_type=jax.ShapeDtypeStruct((NW, LANES), jnp.float32),
    mesh=_mesh,
    scratch_types=[
        pltpu.VMEM((CHUNK,), jnp.float32),
        pltpu.VMEM((LANES,), jnp.float32),
    ],
)
def _partial_sums(x_hbm, out_hbm, buf, part):
    wid = lax.axis_index("s") * NC + lax.axis_index("c")
    base = wid * CHUNK
    pltpu.sync_copy(x_hbm.at[pl.ds(base, CHUNK)], buf)

    def body(i, accs):
        off = i * (UNROLL * LANES)
        return tuple(
            accs[j] + buf[pl.ds(off + j * LANES, LANES)]
            for j in range(UNROLL)
        )

    zero = jnp.zeros((LANES,), jnp.float32)
    accs = lax.fori_loop(0, ITERS, body, (zero,) * UNROLL)
    total = accs[0]
    for j in range(1, UNROLL):
        total = total + accs[j]
    part[...] = total
    pltpu.sync_copy(part, out_hbm.at[wid])


@functools.partial(
    pl.kernel,
    out_type=jax.ShapeDtypeStruct((1, 1), jnp.float32),
    mesh=_mesh,
    scratch_types=[
        pltpu.VMEM((NW, LANES), jnp.float32),
        pltpu.VMEM((LANES,), jnp.float32),
    ],
)
def _combine(parts_hbm, out_hbm, buf, res):
    c = lax.axis_index("c")
    s = lax.axis_index("s")

    @pl.when(jnp.logical_and(c == 0, s == 0))
    def _():
        pltpu.sync_copy(parts_hbm, buf)
        total = buf[0, :]
        for i in range(1, NW):
            total = total + buf[i, :]
        scalar = total[0]
        for i in range(1, LANES):
            scalar = scalar + total[i]
        res[...] = jnp.full((LANES,), scalar, jnp.float32)
        pltpu.sync_copy(res.at[pl.ds(0, 1)], out_hbm.at[0])


def kernel(atomic_energy):
    x = atomic_energy.reshape(-1)
    parts = _partial_sums(x)
    return _combine(parts)


# speedup vs baseline: 102.2229x; 1.0066x over previous
import functools

import jax
import jax.numpy as jnp
from jax import lax
from jax.experimental import pallas as pl
from jax.experimental.pallas import tpu as pltpu
from jax.experimental.pallas import tpu_sc as plsc

N = 3200000
NC = 2
NS = 16
NW = NC * NS
CHUNK = N // NW
LANES = 16
NSUB = 10
SUB = CHUNK // NSUB
SVR = SUB // LANES
UNROLL = 5
SITERS = SVR // UNROLL

_mesh = plsc.VectorSubcoreMesh(core_axis_name="c", subcore_axis_name="s")


@functools.partial(
    pl.kernel,
    out_type=jax.ShapeDtypeStruct((NW, LANES), jnp.float32),
    mesh=_mesh,
    scratch_types=[
        pltpu.VMEM((SUB,), jnp.float32),
        pltpu.VMEM((SUB,), jnp.float32),
        pltpu.VMEM((LANES,), jnp.float32),
        pltpu.SemaphoreType.DMA,
        pltpu.SemaphoreType.DMA,
    ],
)
def _partial_sums(x_hbm, out_hbm, buf0, buf1, part, sem0, sem1):
    wid = lax.axis_index("s") * NC + lax.axis_index("c")
    base = wid * CHUNK
    sems = (sem0, sem1)
    bufs = (buf0, buf1)

    def copy(idx, b):
        return pltpu.make_async_copy(
            x_hbm.at[pl.ds(base + idx * SUB, SUB)], bufs[b], sems[b]
        )

    copy(0, 0).start()
    copy(1, 1).start()

    def accum(b, idx, total):
        copy(idx, b).wait()

        def body(i, accs):
            off = i * (UNROLL * LANES)
            return tuple(
                accs[j] + bufs[b][pl.ds(off + j * LANES, LANES)]
                for j in range(UNROLL)
            )

        zero = jnp.zeros((LANES,), jnp.float32)
        accs = lax.fori_loop(0, SITERS, body, (zero,) * UNROLL)

        @pl.when(idx + 2 < NSUB)
        def _():
            copy(idx + 2, b).start()

        for j in range(UNROLL):
            total = total + accs[j]
        return total

    def outer(g, total):
        total = accum(0, g * 2, total)
        total = accum(1, g * 2 + 1, total)
        return total

    total = lax.fori_loop(0, NSUB // 2, outer, jnp.zeros((LANES,), jnp.float32))
    part[...] = total
    pltpu.sync_copy(part, out_hbm.at[wid])


@functools.partial(
    pl.kernel,
    out_type=jax.ShapeDtypeStruct((1, 1), jnp.float32),
    mesh=_mesh,
    scratch_types=[
        pltpu.VMEM((NW, LANES), jnp.float32),
        pltpu.VMEM((LANES,), jnp.float32),
    ],
)
def _combine(parts_hbm, out_hbm, buf, res):
    c = lax.axis_index("c")
    s = lax.axis_index("s")

    @pl.when(jnp.logical_and(c == 0, s == 0))
    def _():
        pltpu.sync_copy(parts_hbm, buf)
        total = buf[0, :]
        for i in range(1, NW):
            total = total + buf[i, :]
        scalar = total[0]
        for i in range(1, LANES):
            scalar = scalar + total[i]
        res[...] = jnp.full((LANES,), scalar, jnp.float32)
        pltpu.sync_copy(res.at[pl.ds(0, 1)], out_hbm.at[0])


def kernel(atomic_energy):
    x = atomic_energy.reshape(-1)
    parts = _partial_sums(x)
    return _combine(parts)
